# baseline (speedup 1.0000x reference)
import functools

import jax
import jax.numpy as jnp
from jax import lax
from jax.experimental import pallas as pl
from jax.experimental.pallas import tpu as pltpu
from jax.experimental.pallas import tpu_sc as plsc

_L = 16
_NW = 32
_C = 19
_OC = 3
_R = 2048
_HALO = 8


@functools.lru_cache(maxsize=None)
def _build(B: int):
    rows_pw = B // _NW
    n_chunks = rows_pw // _R
    n_groups = _R // _L
    mesh = plsc.VectorSubcoreMesh(core_axis_name="c", subcore_axis_name="s")

    @functools.partial(
        pl.kernel,
        mesh=mesh,
        compiler_params=pltpu.CompilerParams(needs_layout_passes=False),
        out_type=jax.ShapeDtypeStruct((B * _OC,), jnp.float32),
        scratch_types=[
            pltpu.VMEM(((_R + _HALO) * _C,), jnp.float32),
            pltpu.VMEM((_R * _OC,), jnp.float32),
        ],
    )
    def sck(x_hbm, o_hbm, in_v, out_v):
        wid = lax.axis_index("s") * 2 + lax.axis_index("c")
        lane = lax.iota(jnp.int32, _L)
        ridx = lane * _C
        oidx = lane * _OC

        def chunk_body(k, carry):
            cb = wid * rows_pw + k * _R
            halo_row = jnp.where(cb == 0, B - _HALO, cb - _HALO)
            pltpu.sync_copy(x_hbm.at[pl.ds(halo_row * _C, _HALO * _C)],
                            in_v.at[pl.ds(0, _HALO * _C)])
            pltpu.sync_copy(x_hbm.at[pl.ds(cb * _C, _R * _C)],
                            in_v.at[pl.ds(_HALO * _C, _R * _C)])

            def group_body(g, c2_):
                wb = _HALO * _C + g * (_L * _C)
                idx = ridx + wb

                def ld(col):
                    return plsc.load_gather(in_v, [idx + col])

                ha_open = ld(0)
                ha_close = ld(1)
                high_diff = ld(8)
                low_diff = ld(9)
                body_diff = ld(10)
                ma = ld(11)
                ma_prev = plsc.load_gather(in_v, [idx + (11 - _C)])
                ma_sig = ld(12)
                rsi = ld(13)
                s_sig = ld(14)
                bb_up = ld(16)
                bb_lo = ld(17)

                body_big = jnp.abs(ha_close - ha_open) > 0.5
                strong_b = ((ha_close > ha_open) & (body_diff > 0.0)
                            & body_big & (high_diff > 0.0) & (low_diff > 0.0))
                strong_br = ((ha_close < ha_open) & (body_diff < 0.0)
                             & body_big & (high_diff < 0.0) & (low_diff < 0.0))
                slope = (ma - ma_prev) / ma
                sp = slope > 0.0
                sn = slope < 0.0
                pp = (ha_close - bb_lo) / (bb_up - bb_lo)
                pp_lo = pp < 0.2
                pp_hi = pp > 0.8
                bb0 = pp_hi & sn
                bb2 = pp_lo & sp
                ob = (rsi > 0.8) & sn
                osd = (rsi < 0.2) & sp
                msig_hi = ma_sig > 0.1
                msig_lo = ma_sig < -0.1
                ssig_hi = s_sig > 0.1
                ssig_lo = s_sig < -0.1
                ma0 = msig_lo & sn
                ma2 = msig_hi & sp
                long_s = strong_b & sp & ((msig_hi & ssig_lo)
                                          | (msig_hi & pp_lo)
                                          | (ssig_lo & pp_lo))
                short_s = strong_br & sn & ((msig_lo & ssig_hi)
                                            | (msig_lo & pp_hi)
                                            | (ssig_hi & pp_hi))
                c0 = (jnp.where(strong_br, 0.7 * 1.2, 0.0)
                      + jnp.where(ma0, 0.7 * 1.5, 0.0)
                      + jnp.where(ob, 0.7 * 1.0, 0.0)
                      + jnp.where(bb0, 0.7 * 1.2, 0.0)
                      + jnp.where(short_s, 0.8 * 1.8, 0.0))
                c2 = (jnp.where(strong_b, 0.7 * 1.2, 0.0)
                      + jnp.where(ma2, 0.7 * 1.5, 0.0)
                      + jnp.where(osd, 0.7 * 1.0, 0.0)
                      + jnp.where(bb2, 0.7 * 1.2, 0.0)
                      + jnp.where(long_s, 0.8 * 1.8, 0.0))
                x0 = 0.8 + 2.0 * c0
                x2 = 0.8 + 2.0 * c2
                m = jnp.maximum(x0, x2)
                e0 = jnp.exp(x0 - m)
                e1 = jnp.exp(0.4 - m)
                e2 = jnp.exp(x2 - m)
                inv = 1.0 / (e0 + e1 + e2)
                out_b = g * (_L * _OC) + oidx
                plsc.store_scatter(out_v, [out_b], e0 * inv)
                plsc.store_scatter(out_v, [out_b + 1], e1 * inv)
                plsc.store_scatter(out_v, [out_b + 2], e2 * inv)
                return c2_

            lax.fori_loop(0, n_groups, group_body, 0, unroll=2)
            pltpu.sync_copy(out_v, o_hbm.at[pl.ds(cb * _OC, _R * _OC)])
            return carry

        lax.fori_loop(0, n_chunks, chunk_body, 0)

    return sck


def kernel(state):
    B, C = state.shape
    flat = state.reshape(B * C)
    out = _build(B)(flat)
    return out.reshape(B, _OC)
